# baseline (speedup 1.0000x reference)
import functools

import jax
import jax.numpy as jnp
from jax import lax
from jax.experimental import pallas as pl
from jax.experimental.pallas import tpu as pltpu
from jax.experimental.pallas import tpu_sc as plsc

C = 256
S = 16384
T = 1024
D = 31
B = 1024

SROWS = 128

NW = 32
RPW = B // NW
L = 16
UNROLL = 8


def _tc_body(x_ref, w_ref, b_ref, s2_ref, g_ref):
    cls = lax.broadcasted_iota(jnp.int32, (C, 128), 0)

    def count(k, acc):
        row = s2_ref[pl.ds(k, 1), :]
        return acc + (cls == row).astype(jnp.float32)

    acc = lax.fori_loop(0, SROWS, count, jnp.zeros((C, 128), jnp.float32))
    cnt = jnp.sum(acc, axis=1, keepdims=True)

    x = x_ref[...]
    w = w_ref[...]
    logits = lax.dot_general(x, w, (((1,), (1,)), ((), ())),
                             preferred_element_type=jnp.float32)
    ci = lax.broadcasted_iota(jnp.int32, (1, C), 1).astype(jnp.float32)
    q = (jnp.log(ci + 2.0) - jnp.log(ci + 1.0)) / jnp.log(float(C) + 1.0)
    adj = logits + b_ref[...] - jnp.log(float(S) * q)
    sp = jnp.maximum(adj, 0.0) + jnp.log1p(jnp.exp(-jnp.abs(adj)))
    part = lax.dot_general(sp, cnt, (((1,), (0,)), ((), ())),
                           preferred_element_type=jnp.float32)
    g_ref[...] = sp - adj * (1.0 / T) + part * (1.0 / T)


def _tc_tables(x, w, b2, s2):
    return pl.pallas_call(
        _tc_body,
        out_shape=jax.ShapeDtypeStruct((B, C), jnp.float32),
    )(x, w, b2, s2)


def _sc_body(g_hbm, labels_hbm, out_hbm, lab_v, g_v, out_v):
    wid = lax.axis_index("s") * 2 + lax.axis_index("c")
    base = wid * RPW
    pltpu.sync_copy(labels_hbm.at[pl.ds(base, RPW), :], lab_v)
    pltpu.sync_copy(g_hbm.at[pl.ds(base, RPW), :], g_v)

    lanes = lax.iota(jnp.int32, L)

    for grp in range(RPW // L):
        def row_body(r16, outvec, grp=grp):
            r = grp * L + r16
            rsplat = jnp.full((L,), 0, jnp.int32) + r

            def inner(j, acc):
                for k in range(UNROLL):
                    pair = lab_v[r, pl.ds((j * UNROLL + k) * 2 * L, 2 * L)]
                    ia, ib = plsc.unpack(pair, format=plsc.PackFormat.INTERLEAVED,
                                         preferred_element_type=jnp.int32)
                    acc = acc + plsc.load_gather(g_v, [rsplat, ia])
                    acc = acc + plsc.load_gather(g_v, [rsplat, ib])
                return acc

            acc = lax.fori_loop(0, T // (2 * L * UNROLL), inner,
                                jnp.zeros((L,), jnp.float32))
            return outvec + jnp.where(lanes == r16, jnp.sum(acc), 0.0)

        outvec = lax.fori_loop(0, L, row_body, jnp.zeros((L,), jnp.float32))
        out_v[pl.ds(grp * L, L)] = outvec

    pltpu.sync_copy(out_v, out_hbm.at[pl.ds(base, RPW)])


_sc_true_sum = functools.partial(
    pl.kernel,
    out_type=jax.ShapeDtypeStruct((B,), jnp.float32),
    mesh=plsc.VectorSubcoreMesh(core_axis_name="c", subcore_axis_name="s"),
    compiler_params=pltpu.CompilerParams(use_tc_tiling_on_sc=False,
                                         needs_layout_passes=False,
                                         skip_device_barrier=True),
    scratch_types=[
        pltpu.VMEM((RPW, T), jnp.int16),
        pltpu.VMEM((RPW, C), jnp.float32),
        pltpu.VMEM((RPW,), jnp.float32),
    ],
)(_sc_body)


def kernel(inputs, labels, w, b):
    labels = labels.astype(jnp.int16)
    b2 = b.reshape(1, C)
    u = jax.random.uniform(jax.random.key(42), (S,), dtype=jnp.float32)
    sampled = jnp.clip((jnp.exp(u * jnp.log(float(C) + 1.0)) - 1.0)
                       .astype(jnp.int32), 0, C - 1)
    g = _tc_tables(inputs, w, b2, sampled.reshape(SROWS, 128))
    return _sc_true_sum(g, labels)


# speedup vs baseline: 1.1337x; 1.1337x over previous
import functools

import jax
import jax.numpy as jnp
from jax import lax
from jax.experimental import pallas as pl
from jax.experimental.pallas import tpu as pltpu
from jax.experimental.pallas import tpu_sc as plsc

C = 256
S = 16384
T = 1024
D = 31
B = 1024

SROWS = 128

NW = 32
RPW = B // NW
L = 16
UNROLL = 8


def _tc_body(x_ref, w_ref, b_ref, cnt_ref, g_ref):
    cnt = cnt_ref[...]
    x = x_ref[...]
    w = w_ref[...]
    logits = lax.dot_general(x, w, (((1,), (1,)), ((), ())),
                             preferred_element_type=jnp.float32)
    ci = lax.broadcasted_iota(jnp.int32, (1, C), 1).astype(jnp.float32)
    q = (jnp.log(ci + 2.0) - jnp.log(ci + 1.0)) / jnp.log(float(C) + 1.0)
    adj = logits + b_ref[...] - jnp.log(float(S) * q)
    sp = jnp.maximum(adj, 0.0) + jnp.log1p(jnp.exp(-jnp.abs(adj)))
    part = lax.dot_general(sp, cnt, (((1,), (0,)), ((), ())),
                           preferred_element_type=jnp.float32)
    g_ref[...] = sp - adj * (1.0 / T) + part * (1.0 / T)


def _tc_tables(x, w, b2, cnt):
    return pl.pallas_call(
        _tc_body,
        out_shape=jax.ShapeDtypeStruct((B, C), jnp.float32),
    )(x, w, b2, cnt)


def _sc_body(g_hbm, labels_hbm, out_hbm, lab_v, g_v, out_v):
    wid = lax.axis_index("s") * 2 + lax.axis_index("c")
    base = wid * RPW
    pltpu.sync_copy(labels_hbm.at[pl.ds(base, RPW), :], lab_v)
    pltpu.sync_copy(g_hbm.at[pl.ds(base, RPW), :], g_v)

    lanes = lax.iota(jnp.int32, L)

    for grp in range(RPW // L):
        def row_body(r16, outvec, grp=grp):
            r = grp * L + r16
            rsplat = jnp.full((L,), 0, jnp.int32) + r

            def inner(j, acc):
                for k in range(UNROLL):
                    idx = lab_v[r, pl.ds((j * UNROLL + k) * L, L)]
                    acc = acc + plsc.load_gather(g_v, [rsplat, idx])
                return acc

            acc = lax.fori_loop(0, T // (L * UNROLL), inner,
                                jnp.zeros((L,), jnp.float32))
            return outvec + jnp.where(lanes == r16, jnp.sum(acc), 0.0)

        outvec = lax.fori_loop(0, L, row_body, jnp.zeros((L,), jnp.float32))
        out_v[pl.ds(grp * L, L)] = outvec

    pltpu.sync_copy(out_v, out_hbm.at[pl.ds(base, RPW)])


_sc_true_sum = functools.partial(
    pl.kernel,
    out_type=jax.ShapeDtypeStruct((B,), jnp.float32),
    mesh=plsc.VectorSubcoreMesh(core_axis_name="c", subcore_axis_name="s"),
    compiler_params=pltpu.CompilerParams(use_tc_tiling_on_sc=False,
                                         needs_layout_passes=False,
                                         skip_device_barrier=True),
    scratch_types=[
        pltpu.VMEM((RPW, T), jnp.int32),
        pltpu.VMEM((RPW, C), jnp.float32),
        pltpu.VMEM((RPW,), jnp.float32),
    ],
)(_sc_body)


def kernel(inputs, labels, w, b):
    labels = labels.astype(jnp.int32)
    b2 = b.reshape(1, C)
    u = jax.random.uniform(jax.random.key(42), (S,), dtype=jnp.float32)
    sampled = jnp.clip((jnp.exp(u * jnp.log(float(C) + 1.0)) - 1.0)
                       .astype(jnp.int32), 0, C - 1)
    cnt = jnp.sum((sampled[None, :] == jnp.arange(C, dtype=jnp.int32)[:, None])
                  .astype(jnp.float32), axis=1, keepdims=True)
    g = _tc_tables(inputs, w, b2, cnt)
    return _sc_true_sum(g, labels)


# speedup vs baseline: 1.2273x; 1.0825x over previous
import functools

import numpy as np

import jax
import jax.numpy as jnp
from jax import lax
from jax.experimental import pallas as pl
from jax.experimental.pallas import tpu as pltpu
from jax.experimental.pallas import tpu_sc as plsc

C = 256
S = 16384
T = 1024
D = 31
B = 1024

SROWS = 128

NW = 32
RPW = B // NW
L = 16
UNROLL = 8


def _np_sampled_counts() -> np.ndarray:
    def rotl(x, r):
        return ((x << np.uint32(r)) | (x >> np.uint32(32 - r))).astype(np.uint32)

    ks = [np.uint32(0), np.uint32(42), np.uint32(0x1BD11BDA) ^ np.uint32(42)]
    x0 = np.zeros(S, np.uint32) + ks[0]
    x1 = (np.arange(S, dtype=np.uint32) + ks[1]).astype(np.uint32)
    rotations = [(13, 15, 26, 6), (17, 29, 16, 24)]
    for i in range(5):
        for r in rotations[i % 2]:
            x0 = (x0 + x1).astype(np.uint32)
            x1 = rotl(x1, r) ^ x0
        x0 = (x0 + ks[(i + 1) % 3]).astype(np.uint32)
        x1 = (x1 + ks[(i + 2) % 3] + np.uint32(i + 1)).astype(np.uint32)
    bits = x0 ^ x1
    u = (((bits >> np.uint32(9)) | np.uint32(0x3F800000)).view(np.float32)
         - np.float32(1.0))
    ids = np.clip((np.exp(u * np.log(np.float32(C) + 1.0)) - 1.0)
                  .astype(np.int32), 0, C - 1)
    return np.bincount(ids, minlength=C).astype(np.float32).reshape(C, 1)


_CNT = _np_sampled_counts()


def _tc_body(x_ref, w_ref, b_ref, cnt_ref, g_ref):
    cnt = cnt_ref[...]
    x = x_ref[...]
    w = w_ref[...]
    logits = lax.dot_general(x, w, (((1,), (1,)), ((), ())),
                             preferred_element_type=jnp.float32)
    ci = lax.broadcasted_iota(jnp.int32, (1, C), 1).astype(jnp.float32)
    q = (jnp.log(ci + 2.0) - jnp.log(ci + 1.0)) / jnp.log(float(C) + 1.0)
    adj = logits + b_ref[...] - jnp.log(float(S) * q)
    sp = jnp.maximum(adj, 0.0) + jnp.log1p(jnp.exp(-jnp.abs(adj)))
    part = lax.dot_general(sp, cnt, (((1,), (0,)), ((), ())),
                           preferred_element_type=jnp.float32)
    g_ref[...] = sp - adj * (1.0 / T) + part * (1.0 / T)


def _tc_tables(x, w, b2, cnt):
    return pl.pallas_call(
        _tc_body,
        out_shape=jax.ShapeDtypeStruct((B, C), jnp.float32),
    )(x, w, b2, cnt)


def _sc_body(g_hbm, labels_hbm, out_hbm, lab_v, g_v, out_v):
    wid = lax.axis_index("s") * 2 + lax.axis_index("c")
    base = wid * RPW
    pltpu.sync_copy(labels_hbm.at[pl.ds(base, RPW), :], lab_v)
    pltpu.sync_copy(g_hbm.at[pl.ds(base, RPW), :], g_v)

    lanes = lax.iota(jnp.int32, L)

    for grp in range(RPW // L):
        def row_body(r16, outvec, grp=grp):
            r = grp * L + r16
            rsplat = jnp.full((L,), 0, jnp.int32) + r

            def inner(j, acc):
                for k in range(UNROLL):
                    idx = lab_v[r, pl.ds((j * UNROLL + k) * L, L)]
                    acc = acc + plsc.load_gather(g_v, [rsplat, idx])
                return acc

            acc = lax.fori_loop(0, T // (L * UNROLL), inner,
                                jnp.zeros((L,), jnp.float32))
            return outvec + jnp.where(lanes == r16, jnp.sum(acc), 0.0)

        outvec = lax.fori_loop(0, L, row_body, jnp.zeros((L,), jnp.float32))
        out_v[pl.ds(grp * L, L)] = outvec

    pltpu.sync_copy(out_v, out_hbm.at[pl.ds(base, RPW)])


_sc_true_sum = functools.partial(
    pl.kernel,
    out_type=jax.ShapeDtypeStruct((B,), jnp.float32),
    mesh=plsc.VectorSubcoreMesh(core_axis_name="c", subcore_axis_name="s"),
    compiler_params=pltpu.CompilerParams(use_tc_tiling_on_sc=False,
                                         needs_layout_passes=False,
                                         skip_device_barrier=True),
    scratch_types=[
        pltpu.VMEM((RPW, T), jnp.int32),
        pltpu.VMEM((RPW, C), jnp.float32),
        pltpu.VMEM((RPW,), jnp.float32),
    ],
)(_sc_body)


def kernel(inputs, labels, w, b):
    labels = labels.astype(jnp.int32)
    b2 = b.reshape(1, C)
    g = _tc_tables(inputs, w, b2, jnp.asarray(_CNT))
    return _sc_true_sum(g, labels)
